# initial kernel scaffold (unmeasured)
import jax
import jax.numpy as jnp
from jax import lax
from jax.experimental import pallas as pl
from jax.experimental.pallas import tpu as pltpu


def kernel(
    x,
):
    def body(*refs):
        pass

    out_shape = jax.ShapeDtypeStruct(..., jnp.float32)
    return pl.pallas_call(body, out_shape=out_shape)(...)


# baseline (device time: 4379823 ns/iter reference)
import jax
import jax.numpy as jnp
from jax import lax
from jax.experimental import pallas as pl
from jax.experimental.pallas import tpu as pltpu

N_Z = 4


def kernel(x):
    m_per, n = x.shape

    def body(x_ref, out_ref, local_sem, send_sems, recv_sems):
        my_x = lax.axis_index("x")
        my_y = lax.axis_index("y")
        my_z = lax.axis_index("z")
        left = (my_z - 1) % N_Z
        right = (my_z + 1) % N_Z

        barrier_sem = pltpu.get_barrier_semaphore()
        for nbr in [left, right]:
            pl.semaphore_signal(
                barrier_sem,
                inc=1,
                device_id=(my_x, my_y, nbr),
                device_id_type=pl.DeviceIdType.MESH,
            )
        pl.semaphore_wait(barrier_sem, 2)

        cp = pltpu.make_async_copy(
            x_ref, out_ref.at[pl.ds(my_z * m_per, m_per), :], local_sem
        )
        cp.start()
        cp.wait()

        for h in range(N_Z - 1):
            origin = (my_z - h) % N_Z
            rdma = pltpu.make_async_remote_copy(
                src_ref=out_ref.at[pl.ds(origin * m_per, m_per), :],
                dst_ref=out_ref.at[pl.ds(origin * m_per, m_per), :],
                send_sem=send_sems.at[h],
                recv_sem=recv_sems.at[h],
                device_id=(my_x, my_y, right),
                device_id_type=pl.DeviceIdType.MESH,
            )
            rdma.start()
            rdma.wait()

    return pl.pallas_call(
        body,
        out_shape=jax.ShapeDtypeStruct((N_Z * m_per, n), x.dtype),
        in_specs=[pl.BlockSpec(memory_space=pl.ANY)],
        out_specs=pl.BlockSpec(memory_space=pl.ANY),
        scratch_shapes=[
            pltpu.SemaphoreType.DMA,
            pltpu.SemaphoreType.DMA((N_Z - 1,)),
            pltpu.SemaphoreType.DMA((N_Z - 1,)),
        ],
        compiler_params=pltpu.CompilerParams(collective_id=0),
    )(x)


# device time: 1175864 ns/iter; 3.7248x vs baseline; 3.7248x over previous
import jax
import jax.numpy as jnp
from jax import lax
from jax.experimental import pallas as pl
from jax.experimental.pallas import tpu as pltpu

N_Z = 4
MESH = pl.DeviceIdType.MESH


def kernel(x):
    m_per, n = x.shape
    Q = m_per // 4
    H = Q // 2

    def body(
        x_ref, out_ref, vmem_ref, loc_sems,
        zs_r, zr_r, zs_l, zr_l,
        pfs, pfr, pbs, pbr,
        hfs, hfr, hbs, hbr,
    ):
        my_x = lax.axis_index("x")
        my_y = lax.axis_index("y")
        my_z = lax.axis_index("z")

        p = 2 * my_x + jnp.where(my_x == 0, my_y, 1 - my_y)
        pn = (p + 1) % 4
        pp = (p + 3) % 4

        def pcoords(q):
            return (
                jnp.where(q >= 2, 1, 0),
                jnp.where((q == 1) | (q == 2), 1, 0),
            )

        nxt = pcoords(pn) + (my_z,)
        prv = pcoords(pp) + (my_z,)
        zrt = (my_x, my_y, jnp.minimum(my_z + 1, N_Z - 1))
        zlt = (my_x, my_y, jnp.maximum(my_z - 1, 0))
        me = (my_x, my_y, my_z)

        def slot(k, q):
            return k * m_per + q * Q

        barrier = pltpu.get_barrier_semaphore()
        for tgt in (nxt, prv):
            pl.semaphore_signal(barrier, inc=1, device_id=tgt,
                                device_id_type=MESH)

        @pl.when(my_z < N_Z - 1)
        def _():
            pl.semaphore_signal(barrier, inc=1, device_id=zrt,
                                device_id_type=MESH)

        @pl.when(my_z > 0)
        def _():
            pl.semaphore_signal(barrier, inc=1, device_id=zlt,
                                device_id_type=MESH)

        pl.semaphore_wait(barrier, 2)

        @pl.when(my_z < N_Z - 1)
        def _():
            pl.semaphore_wait(barrier, 1)

        @pl.when(my_z > 0)
        def _():
            pl.semaphore_wait(barrier, 1)

        @pl.when(my_z < N_Z - 1)
        def _():
            pltpu.make_async_remote_copy(
                src_ref=x_ref.at[pl.ds(p * Q, Q), :],
                dst_ref=out_ref.at[pl.ds(slot(my_z, p), Q), :],
                send_sem=zs_r.at[0], recv_sem=zr_r.at[0],
                device_id=zrt, device_id_type=MESH,
            ).start()

        @pl.when(my_z > 0)
        def _():
            pltpu.make_async_remote_copy(
                src_ref=x_ref.at[pl.ds(p * Q, Q), :],
                dst_ref=out_ref.at[pl.ds(slot(my_z, p), Q), :],
                send_sem=zs_l.at[0], recv_sem=zr_l.at[0],
                device_id=zlt, device_id_type=MESH,
            ).start()

        ins = [
            pltpu.make_async_copy(
                x_ref.at[pl.ds(i * Q, Q), :],
                vmem_ref.at[i % 2],
                loc_sems.at[i % 2],
            )
            for i in range(4)
        ]
        outs = [None] * 4
        ins[0].start()
        for i in range(4):
            ins[i].wait()
            outs[i] = pltpu.make_async_copy(
                vmem_ref.at[i % 2],
                out_ref.at[pl.ds(slot(my_z, i), Q), :],
                loc_sems.at[2 + i % 2],
            )
            outs[i].start()
            if i + 1 < 4:
                if i - 1 >= 0:
                    outs[i - 1].wait()
                ins[i + 1].start()
        outs[2].wait()
        outs[3].wait()

        def sides(s):
            return (
                (0, my_z - 1 - s, my_z - 1 - s >= 0),
                (1, my_z + 1 + s, my_z + 1 + s <= N_Z - 1),
            )

        for s in range(3):
            for side, k_expr, cond in sides(s):
                @pl.when(cond)
                def _(s=s, side=side, k=k_expr):
                    rsem = zr_r if side == 0 else zr_l
                    ssem = zs_r if side == 0 else zs_l
                    piece = out_ref.at[pl.ds(slot(k, p), Q), :]
                    pltpu.make_async_remote_copy(
                        src_ref=piece, dst_ref=piece,
                        send_sem=ssem.at[s], recv_sem=rsem.at[s],
                        device_id=me, device_id_type=MESH,
                    ).wait_recv()
                    if s + 1 < 3:
                        fcond = (my_z < N_Z - 1) if side == 0 else (my_z > 0)
                        ftgt = zrt if side == 0 else zlt

                        @pl.when(fcond)
                        def _():
                            pltpu.make_async_remote_copy(
                                src_ref=piece, dst_ref=piece,
                                send_sem=ssem.at[s + 1],
                                recv_sem=rsem.at[s + 1],
                                device_id=ftgt, device_id_type=MESH,
                            ).start()
                    idx = side * 3 + s
                    pltpu.make_async_remote_copy(
                        src_ref=piece, dst_ref=piece,
                        send_sem=pfs.at[idx], recv_sem=pfr.at[idx],
                        device_id=nxt, device_id_type=MESH,
                    ).start()
                    pltpu.make_async_remote_copy(
                        src_ref=piece, dst_ref=piece,
                        send_sem=pbs.at[idx], recv_sem=pbr.at[idx],
                        device_id=prv, device_id_type=MESH,
                    ).start()

        for s in range(3):
            for side, k_expr, cond in sides(s):
                @pl.when(cond)
                def _(s=s, side=side, k=k_expr):
                    idx = side * 3 + s
                    fr_piece = out_ref.at[pl.ds(slot(k, pp), Q), :]
                    pltpu.make_async_remote_copy(
                        src_ref=fr_piece, dst_ref=fr_piece,
                        send_sem=pfs.at[idx], recv_sem=pfr.at[idx],
                        device_id=me, device_id_type=MESH,
                    ).wait_recv()
                    top = out_ref.at[pl.ds(slot(k, pp), H), :]
                    pltpu.make_async_remote_copy(
                        src_ref=top, dst_ref=top,
                        send_sem=hfs.at[idx], recv_sem=hfr.at[idx],
                        device_id=nxt, device_id_type=MESH,
                    ).start()
                    br_piece = out_ref.at[pl.ds(slot(k, pn), Q), :]
                    pltpu.make_async_remote_copy(
                        src_ref=br_piece, dst_ref=br_piece,
                        send_sem=pbs.at[idx], recv_sem=pbr.at[idx],
                        device_id=me, device_id_type=MESH,
                    ).wait_recv()
                    bot = out_ref.at[pl.ds(slot(k, pn) + H, H), :]
                    pltpu.make_async_remote_copy(
                        src_ref=bot, dst_ref=bot,
                        send_sem=hbs.at[idx], recv_sem=hbr.at[idx],
                        device_id=prv, device_id_type=MESH,
                    ).start()

        for s in range(3):
            for side, k_expr, cond in sides(s):
                @pl.when(cond)
                def _(s=s, side=side, k=k_expr):
                    idx = side * 3 + s
                    po = (p + 2) % 4
                    otop = out_ref.at[pl.ds(slot(k, po), H), :]
                    pltpu.make_async_remote_copy(
                        src_ref=otop, dst_ref=otop,
                        send_sem=hfs.at[idx], recv_sem=hfr.at[idx],
                        device_id=me, device_id_type=MESH,
                    ).wait_recv()
                    obot = out_ref.at[pl.ds(slot(k, po) + H, H), :]
                    pltpu.make_async_remote_copy(
                        src_ref=obot, dst_ref=obot,
                        send_sem=hbs.at[idx], recv_sem=hbr.at[idx],
                        device_id=me, device_id_type=MESH,
                    ).wait_recv()
                    piece = out_ref.at[pl.ds(slot(k, p), Q), :]
                    pltpu.make_async_remote_copy(
                        src_ref=piece, dst_ref=piece,
                        send_sem=pfs.at[idx], recv_sem=pfr.at[idx],
                        device_id=me, device_id_type=MESH,
                    ).wait_send()
                    pltpu.make_async_remote_copy(
                        src_ref=piece, dst_ref=piece,
                        send_sem=pbs.at[idx], recv_sem=pbr.at[idx],
                        device_id=me, device_id_type=MESH,
                    ).wait_send()
                    topd = out_ref.at[pl.ds(slot(k, pp), H), :]
                    pltpu.make_async_remote_copy(
                        src_ref=topd, dst_ref=topd,
                        send_sem=hfs.at[idx], recv_sem=hfr.at[idx],
                        device_id=me, device_id_type=MESH,
                    ).wait_send()
                    botd = out_ref.at[pl.ds(slot(k, pn) + H, H), :]
                    pltpu.make_async_remote_copy(
                        src_ref=botd, dst_ref=botd,
                        send_sem=hbs.at[idx], recv_sem=hbr.at[idx],
                        device_id=me, device_id_type=MESH,
                    ).wait_send()

        for s in range(3):
            @pl.when((my_z - s >= 0) & (my_z < N_Z - 1))
            def _(s=s):
                k = my_z - s
                piece = out_ref.at[pl.ds(slot(k, p), Q), :]
                pltpu.make_async_remote_copy(
                    src_ref=piece, dst_ref=piece,
                    send_sem=zs_r.at[s], recv_sem=zr_r.at[s],
                    device_id=me, device_id_type=MESH,
                ).wait_send()

            @pl.when((my_z + s <= N_Z - 1) & (my_z > 0))
            def _(s=s):
                k = my_z + s
                piece = out_ref.at[pl.ds(slot(k, p), Q), :]
                pltpu.make_async_remote_copy(
                    src_ref=piece, dst_ref=piece,
                    send_sem=zs_l.at[s], recv_sem=zr_l.at[s],
                    device_id=me, device_id_type=MESH,
                ).wait_send()

    return pl.pallas_call(
        body,
        out_shape=jax.ShapeDtypeStruct((N_Z * m_per, n), x.dtype),
        in_specs=[pl.BlockSpec(memory_space=pl.ANY)],
        out_specs=pl.BlockSpec(memory_space=pl.ANY),
        scratch_shapes=[
            pltpu.VMEM((2, Q, n), jnp.float32),
            pltpu.SemaphoreType.DMA((4,)),
            pltpu.SemaphoreType.DMA((3,)),
            pltpu.SemaphoreType.DMA((3,)),
            pltpu.SemaphoreType.DMA((3,)),
            pltpu.SemaphoreType.DMA((3,)),
            pltpu.SemaphoreType.DMA((6,)),
            pltpu.SemaphoreType.DMA((6,)),
            pltpu.SemaphoreType.DMA((6,)),
            pltpu.SemaphoreType.DMA((6,)),
            pltpu.SemaphoreType.DMA((6,)),
            pltpu.SemaphoreType.DMA((6,)),
            pltpu.SemaphoreType.DMA((6,)),
            pltpu.SemaphoreType.DMA((6,)),
        ],
        compiler_params=pltpu.CompilerParams(collective_id=0),
    )(x)
